# baseline (speedup 1.0000x reference)
import functools

import jax
import jax.numpy as jnp
from jax import lax
from jax.experimental import pallas as pl
from jax.experimental.pallas import tpu as pltpu
from jax.experimental.pallas import tpu_sc as plsc

N = 10000
K = 20
HID = 32
EC = 8
F32 = jnp.float32
EPS = 1e-5


_RB = 1000
_CW = 1024
_NCH = N // _CW + 1


def _knn_body(bounds, fs_blk, fsT_c, brow, bcol_c, out_s, out_p,
              rtv_s, rti_s, rtv_p, rti_p):
    rb = pl.program_id(0)
    c = pl.program_id(1)
    cs = bounds[rb, 0]
    ce = bounds[rb, 1]
    wstart = c * _CW
    active = ((wstart < ce) & (wstart + _CW > cs)) | (c == 0)
    last_c = jnp.maximum((ce - 1) // _CW, 0)

    @pl.when(active)
    def _work():
        @pl.when(c == 0)
        def _seed():
            for rtv, rti in ((rtv_s, rti_s), (rtv_p, rti_p)):
                rtv[...] = jnp.full((_RB, K), float("inf"), F32)
                rti[...] = jnp.zeros((_RB, K), F32)

        same = brow[...] == bcol_c[...]
        in_rng = lax.broadcasted_iota(jnp.int32, (1, _CW), 1) + wstart < N
        ok = same & in_rng
        cols = lax.broadcasted_iota(jnp.int32, (_RB, K + _CW), 1)
        colsK = lax.broadcasted_iota(jnp.int32, (_RB, K), 1)
        fsb = fs_blk[...]
        fsT = fsT_c[...]
        for lo, (rtv, rti, out) in ((0, (rtv_s, rti_s, out_s)),
                                    (EC, (rtv_p, rti_p, out_p))):
            ft_r = fsb[:, lo:lo + EC]
            ftT = fsT[lo:lo + EC, :]
            mm = lax.dot_general(ft_r, ftT, (((1,), (0,)), ((), ())),
                                 preferred_element_type=F32)
            sq_r = jnp.sum(ft_r * ft_r, axis=1, keepdims=True)
            sq_c = jnp.sum(ftT * ftT, axis=0, keepdims=True)
            d2m = jnp.where(ok, sq_r + sq_c - 2.0 * mm, 1e30)
            catv = jnp.concatenate([rtv[...], d2m], axis=1)
            rtiv = rti[...]
            vals, idxs = [], []
            for _ in range(K):
                m = jnp.min(catv, axis=1, keepdims=True)
                cand = jnp.where(catv == m, cols, 2 ** 30)
                a = jnp.min(cand, axis=1, keepdims=True)
                old = jnp.sum(jnp.where(colsK == a, rtiv, 0.0),
                              axis=1, keepdims=True)
                gidx = jnp.where(a < K, old,
                                 (a - K + wstart).astype(F32))
                vals.append(m)
                idxs.append(gidx)
                catv = jnp.where(cols == a, float("inf"), catv)
            newi = jnp.concatenate(idxs, axis=1)
            rtv[...] = jnp.concatenate(vals, axis=1)
            rti[...] = newi

            @pl.when(c == last_c)
            def _emit():
                out[...] = newi.astype(jnp.int32)


def _knn2(hs, hp, batch_row, batch_col, bounds):
    fs = jnp.concatenate([hs, hp], axis=1)
    fsT = jnp.concatenate(
        [fs.T, jnp.zeros((2 * EC, _NCH * _CW - N), F32)], axis=1)
    bcolp = jnp.concatenate(
        [batch_col, jnp.full((1, _NCH * _CW - N), -1, jnp.int32)], axis=1)
    kspec = pl.BlockSpec((_RB, K), lambda i, c: (i, 0))
    return pl.pallas_call(
        _knn_body,
        grid=(N // _RB, _NCH),
        in_specs=[
            pl.BlockSpec(memory_space=pltpu.SMEM),
            pl.BlockSpec((_RB, 2 * EC), lambda i, c: (i, 0)),
            pl.BlockSpec((2 * EC, _CW), lambda i, c: (0, c)),
            pl.BlockSpec((_RB, 1), lambda i, c: (i, 0)),
            pl.BlockSpec((1, _CW), lambda i, c: (0, c)),
        ],
        out_specs=(kspec, kspec),
        out_shape=(jax.ShapeDtypeStruct((N, K), jnp.int32),
                   jax.ShapeDtypeStruct((N, K), jnp.int32)),
        scratch_shapes=[pltpu.VMEM((_RB, K), F32)] * 4,
    )(bounds, fs, fsT, batch_row, bcolp)


def _sc_gather(table_pad, idx_pad, b_pad, b_per_w, nc):
    mesh = plsc.VectorSubcoreMesh(core_axis_name="c", subcore_axis_name="s")

    @functools.partial(
        pl.kernel, mesh=mesh,
        out_type=jax.ShapeDtypeStruct((b_pad, 16), F32),
        compiler_params=pltpu.CompilerParams(use_tc_tiling_on_sc=False),
        scratch_types=[
            pltpu.VMEM((b_per_w,), jnp.int32),
            pltpu.VMEM((b_per_w, 16), F32),
            pltpu.SemaphoreType.DMA,
        ],
    )
    def gk(table_hbm, idx_hbm, out_hbm, idx_v, rows_v, sem):
        wid = lax.axis_index("s") * nc + lax.axis_index("c")
        base = wid * b_per_w
        pltpu.sync_copy(idx_hbm.at[pl.ds(base, b_per_w)], idx_v)
        pltpu.async_copy(table_hbm.at[idx_v], rows_v, sem).wait()
        pltpu.sync_copy(rows_v, out_hbm.at[pl.ds(base, b_per_w)])

    return gk(table_pad, idx_pad)


_NEDGE = N * K


def _edge_body(hs, g2, w1a, w1b, b1, g1, be1, w2, b2, g2w, be2, out, stats):
    p = pl.program_id(0)
    k = pl.program_id(1)
    inv_n = jnp.float32(1.0 / _NEDGE)

    @pl.when((p == 0) & (k == 0))
    def _init():
        stats[...] = jnp.zeros((8, HID), F32)

    @pl.when((p == 1) & (k == 0))
    def _fin1():
        m = stats[0:1, :] * inv_n
        var = stats[1:2, :] * inv_n - m * m
        a = g1[...] / jnp.sqrt(var + EPS)
        stats[4:5, :] = a
        stats[5:6, :] = be1[...] - m * a

    @pl.when((p == 2) & (k == 0))
    def _fin2():
        m = stats[2:3, :] * inv_n
        var = stats[3:4, :] * inv_n - m * m
        a = g2w[...] / jnp.sqrt(var + EPS)
        stats[6:7, :] = a
        stats[7:8, :] = be2[...] - m * a

    xi = hs[...]
    xj = g2[0, :, 0:EC]
    d = xj - xi
    y1 = (lax.dot_general(xi, w1a[...], (((1,), (0,)), ((), ())),
                          preferred_element_type=F32)
          + lax.dot_general(d, w1b[...], (((1,), (0,)), ((), ())),
                            preferred_element_type=F32)
          + b1[...])

    @pl.when(p == 0)
    def _acc1():
        stats[0:1, :] = stats[0:1, :] + jnp.sum(y1, axis=0, keepdims=True)
        stats[1:2, :] = stats[1:2, :] + jnp.sum(y1 * y1, axis=0, keepdims=True)

    h1 = jax.nn.relu(y1 * stats[4:5, :] + stats[5:6, :])
    y2 = lax.dot_general(h1, w2[...], (((1,), (0,)), ((), ())),
                         preferred_element_type=F32) + b2[...]

    @pl.when(p == 1)
    def _acc2():
        stats[2:3, :] = stats[2:3, :] + jnp.sum(y2, axis=0, keepdims=True)
        stats[3:4, :] = stats[3:4, :] + jnp.sum(y2 * y2, axis=0, keepdims=True)

    @pl.when(p == 2)
    def _store():
        h2 = jax.nn.relu(y2 * stats[6:7, :] + stats[7:8, :])

        @pl.when(k == 0)
        def _first():
            out[...] = h2

        @pl.when(k > 0)
        def _rest():
            out[...] = jnp.maximum(out[...], h2)


def _edge_conv(hs, g2, w1, b1, g1, be1, w2, b2, g2w, be2):
    w1a = w1[:EC, :]
    w1b = w1[EC:, :]
    row = lambda v: v.reshape(1, -1)
    return pl.pallas_call(
        _edge_body,
        grid=(3, K),
        in_specs=[
            pl.BlockSpec((N, EC), lambda p, k: (0, 0)),
            pl.BlockSpec((1, N, 16), lambda p, k: (k, 0, 0)),
            pl.BlockSpec((EC, HID), lambda p, k: (0, 0)),
            pl.BlockSpec((EC, HID), lambda p, k: (0, 0)),
            pl.BlockSpec((1, HID), lambda p, k: (0, 0)),
            pl.BlockSpec((1, HID), lambda p, k: (0, 0)),
            pl.BlockSpec((1, HID), lambda p, k: (0, 0)),
            pl.BlockSpec((HID, HID), lambda p, k: (0, 0)),
            pl.BlockSpec((1, HID), lambda p, k: (0, 0)),
            pl.BlockSpec((1, HID), lambda p, k: (0, 0)),
            pl.BlockSpec((1, HID), lambda p, k: (0, 0)),
        ],
        out_specs=pl.BlockSpec((N, HID), lambda p, k: (0, 0)),
        out_shape=jax.ShapeDtypeStruct((N, HID), F32),
        scratch_shapes=[pltpu.VMEM((8, HID), F32)],
    )(hs, g2, w1a, w1b, row(b1), row(g1), row(be1), w2, row(b2), row(g2w),
      row(be2))


def _cls_body(os_, op_, wca, wcb, bc1, gc, bec, wc2, bc2, out):
    y = (lax.dot_general(os_[...], wca[...], (((1,), (0,)), ((), ())),
                         preferred_element_type=F32)
         + lax.dot_general(op_[...], wcb[...], (((1,), (0,)), ((), ())),
                           preferred_element_type=F32)
         + bc1[...])
    m = jnp.mean(y, axis=0, keepdims=True)
    v = jnp.mean((y - m) ** 2, axis=0, keepdims=True)
    h = jax.nn.relu((y - m) / jnp.sqrt(v + EPS) * gc[...] + bec[...])
    out[...] = lax.dot_general(h, wc2[...], (((1,), (0,)), ((), ())),
                               preferred_element_type=F32) + bc2[...]


def _classifier(o_s, o_p, wc1, bc1, gc, bec, wc2, bc2):
    row = lambda v: v.reshape(1, -1)
    return pl.pallas_call(
        _cls_body,
        out_shape=jax.ShapeDtypeStruct((N, 1), F32),
    )(o_s, o_p, wc1[:HID, :], wc1[HID:, :], row(bc1), row(gc), row(bec),
      wc2, row(bc2))


def _bn_host(x, g, b):
    m = jnp.mean(x, axis=0, keepdims=True)
    v = jnp.mean((x - m) ** 2, axis=0, keepdims=True)
    return (x - m) / jnp.sqrt(v + EPS) * g + b


def _gather_edges(feat, idx):
    try:
        info = plsc.get_sparse_core_info()
        nc, ns = info.num_cores, info.num_subcores
    except Exception:
        nc, ns = 2, 16
    nw = nc * ns
    align = 8 * nw
    b_pad = ((_NEDGE + align - 1) // align) * align
    b_per_w = b_pad // nw
    table = jnp.concatenate([feat, jnp.zeros((N, 16 - EC), F32)], axis=1)
    idx_flat = idx.T.reshape(-1)
    idx_pad = jnp.concatenate(
        [idx_flat, jnp.zeros((b_pad - _NEDGE,), jnp.int32)])
    g = _sc_gather(table, idx_pad, b_pad, b_per_w, nc)
    return lax.slice(g, (0, 0), (_NEDGE, 16)).reshape(K, N, 16)


def kernel(x, batch, g_sn, b_sn, g_tn, b_tn, W_se, bi_se, g_se, be_se, W_pe,
           bi_pe, g_pe, be_pe, W1s, b1s, g1s, be1s, W2s, b2s, g2s, be2s, W1p,
           b1p, g1p, be1p, W2p, b2p, g2p, be2p, Wc1, bc1, gc, bec, Wc2, bc2):
    pos = x[:, :3]
    xs = _bn_host(pos, g_sn, b_sn)
    xt = _bn_host(x[:, 3:4], g_tn, b_tn)
    xp = jnp.concatenate([xs, xt], axis=1)
    hs = jax.nn.relu(_bn_host(xs @ W_se + bi_se, g_se, be_se))
    hp = jax.nn.relu(_bn_host(xp @ W_pe + bi_pe, g_pe, be_pe))

    batch_row = batch.reshape(N, 1)
    batch_col = batch.reshape(1, N)
    lo = batch[:: _RB]
    hi = batch[_RB - 1:: _RB]
    starts = jnp.searchsorted(batch, lo, side="left").astype(jnp.int32)
    ends = jnp.searchsorted(batch, hi, side="right").astype(jnp.int32)
    bounds = jnp.stack([starts, ends], axis=1)
    idx_s, idx_p = _knn2(hs, hp, batch_row, batch_col, bounds)

    g2_s = _gather_edges(hs, idx_s)
    g2_p = _gather_edges(hp, idx_p)

    o_s = _edge_conv(hs, g2_s, W1s, b1s, g1s, be1s, W2s, b2s, g2s, be2s)
    o_p = _edge_conv(hp, g2_p, W1p, b1p, g1p, be1p, W2p, b2p, g2p, be2p)

    return _classifier(o_s, o_p, Wc1, bc1, gc, bec, Wc2, bc2)


# speedup vs baseline: 1.3502x; 1.3502x over previous
import functools

import jax
import jax.numpy as jnp
from jax import lax
from jax.experimental import pallas as pl
from jax.experimental.pallas import tpu as pltpu
from jax.experimental.pallas import tpu_sc as plsc

N = 10000
K = 20
HID = 32
EC = 8
F32 = jnp.float32
EPS = 1e-5


_RB = 400
_CW = 512
_NCH = N // _CW + 1


def _knn_body(bounds, fs_blk, fsT_c, brow, bcol_c, out_s, out_p,
              rtv_s, rti_s, rtv_p, rti_p):
    rb = pl.program_id(0)
    c = pl.program_id(1)
    cs = bounds[rb, 0]
    ce = bounds[rb, 1]
    wstart = c * _CW
    active = ((wstart < ce) & (wstart + _CW > cs)) | (c == 0)
    last_c = jnp.maximum((ce - 1) // _CW, 0)

    @pl.when(active)
    def _work():
        @pl.when(c == 0)
        def _seed():
            for rtv, rti in ((rtv_s, rti_s), (rtv_p, rti_p)):
                rtv[...] = jnp.full((_RB, K), float("inf"), F32)
                rti[...] = jnp.zeros((_RB, K), F32)

        same = brow[...] == bcol_c[...]
        in_rng = lax.broadcasted_iota(jnp.int32, (1, _CW), 1) + wstart < N
        ok = same & in_rng
        cols = lax.broadcasted_iota(jnp.int32, (_RB, K + _CW), 1)
        colsK = lax.broadcasted_iota(jnp.int32, (_RB, K), 1)
        fsb = fs_blk[...]
        fsT = fsT_c[...]
        for lo, (rtv, rti, out) in ((0, (rtv_s, rti_s, out_s)),
                                    (EC, (rtv_p, rti_p, out_p))):
            ft_r = fsb[:, lo:lo + EC]
            ftT = fsT[lo:lo + EC, :]
            mm = lax.dot_general(ft_r, ftT, (((1,), (0,)), ((), ())),
                                 preferred_element_type=F32)
            sq_r = jnp.sum(ft_r * ft_r, axis=1, keepdims=True)
            sq_c = jnp.sum(ftT * ftT, axis=0, keepdims=True)
            d2m = jnp.where(ok, sq_r + sq_c - 2.0 * mm, 1e30)
            catv = jnp.concatenate([rtv[...], d2m], axis=1)
            rtiv = rti[...]
            vals, idxs = [], []
            for _ in range(K):
                m = jnp.min(catv, axis=1, keepdims=True)
                cand = jnp.where(catv == m, cols, 2 ** 30)
                a = jnp.min(cand, axis=1, keepdims=True)
                old = jnp.sum(jnp.where(colsK == a, rtiv, 0.0),
                              axis=1, keepdims=True)
                gidx = jnp.where(a < K, old,
                                 (a - K + wstart).astype(F32))
                vals.append(m)
                idxs.append(gidx)
                catv = jnp.where(cols == a, float("inf"), catv)
            newi = jnp.concatenate(idxs, axis=1)
            rtv[...] = jnp.concatenate(vals, axis=1)
            rti[...] = newi

            @pl.when(c == last_c)
            def _emit():
                out[...] = newi.astype(jnp.int32)


def _knn2(hs, hp, batch_row, batch_col, bounds):
    fs = jnp.concatenate([hs, hp], axis=1)
    fsT = jnp.concatenate(
        [fs.T, jnp.zeros((2 * EC, _NCH * _CW - N), F32)], axis=1)
    bcolp = jnp.concatenate(
        [batch_col, jnp.full((1, _NCH * _CW - N), -1, jnp.int32)], axis=1)
    kspec = pl.BlockSpec((_RB, K), lambda i, c: (i, 0))
    return pl.pallas_call(
        _knn_body,
        grid=(N // _RB, _NCH),
        in_specs=[
            pl.BlockSpec(memory_space=pltpu.SMEM),
            pl.BlockSpec((_RB, 2 * EC), lambda i, c: (i, 0)),
            pl.BlockSpec((2 * EC, _CW), lambda i, c: (0, c)),
            pl.BlockSpec((_RB, 1), lambda i, c: (i, 0)),
            pl.BlockSpec((1, _CW), lambda i, c: (0, c)),
        ],
        out_specs=(kspec, kspec),
        out_shape=(jax.ShapeDtypeStruct((N, K), jnp.int32),
                   jax.ShapeDtypeStruct((N, K), jnp.int32)),
        scratch_shapes=[pltpu.VMEM((_RB, K), F32)] * 4,
    )(bounds, fs, fsT, batch_row, bcolp)


def _sc_gather(table_pad, idx_pad, b_pad, b_per_w, nc):
    mesh = plsc.VectorSubcoreMesh(core_axis_name="c", subcore_axis_name="s")

    @functools.partial(
        pl.kernel, mesh=mesh,
        out_type=jax.ShapeDtypeStruct((b_pad, 16), F32),
        compiler_params=pltpu.CompilerParams(use_tc_tiling_on_sc=False),
        scratch_types=[
            pltpu.VMEM((b_per_w,), jnp.int32),
            pltpu.VMEM((b_per_w, 16), F32),
            pltpu.SemaphoreType.DMA,
        ],
    )
    def gk(table_hbm, idx_hbm, out_hbm, idx_v, rows_v, sem):
        wid = lax.axis_index("s") * nc + lax.axis_index("c")
        base = wid * b_per_w
        pltpu.sync_copy(idx_hbm.at[pl.ds(base, b_per_w)], idx_v)
        pltpu.async_copy(table_hbm.at[idx_v], rows_v, sem).wait()
        pltpu.sync_copy(rows_v, out_hbm.at[pl.ds(base, b_per_w)])

    return gk(table_pad, idx_pad)


_NEDGE = N * K


def _edge_body(hs, g2, w1a, w1b, b1, g1, be1, w2, b2, g2w, be2, out, stats):
    p = pl.program_id(0)
    k = pl.program_id(1)
    inv_n = jnp.float32(1.0 / _NEDGE)

    @pl.when((p == 0) & (k == 0))
    def _init():
        stats[...] = jnp.zeros((8, HID), F32)

    @pl.when((p == 1) & (k == 0))
    def _fin1():
        m = stats[0:1, :] * inv_n
        var = stats[1:2, :] * inv_n - m * m
        a = g1[...] / jnp.sqrt(var + EPS)
        stats[4:5, :] = a
        stats[5:6, :] = be1[...] - m * a

    @pl.when((p == 2) & (k == 0))
    def _fin2():
        m = stats[2:3, :] * inv_n
        var = stats[3:4, :] * inv_n - m * m
        a = g2w[...] / jnp.sqrt(var + EPS)
        stats[6:7, :] = a
        stats[7:8, :] = be2[...] - m * a

    xi = hs[...]
    xj = g2[0, :, 0:EC]
    d = xj - xi
    y1 = (lax.dot_general(xi, w1a[...], (((1,), (0,)), ((), ())),
                          preferred_element_type=F32)
          + lax.dot_general(d, w1b[...], (((1,), (0,)), ((), ())),
                            preferred_element_type=F32)
          + b1[...])

    @pl.when(p == 0)
    def _acc1():
        stats[0:1, :] = stats[0:1, :] + jnp.sum(y1, axis=0, keepdims=True)
        stats[1:2, :] = stats[1:2, :] + jnp.sum(y1 * y1, axis=0, keepdims=True)

    h1 = jax.nn.relu(y1 * stats[4:5, :] + stats[5:6, :])
    y2 = lax.dot_general(h1, w2[...], (((1,), (0,)), ((), ())),
                         preferred_element_type=F32) + b2[...]

    @pl.when(p == 1)
    def _acc2():
        stats[2:3, :] = stats[2:3, :] + jnp.sum(y2, axis=0, keepdims=True)
        stats[3:4, :] = stats[3:4, :] + jnp.sum(y2 * y2, axis=0, keepdims=True)

    @pl.when(p == 2)
    def _store():
        h2 = jax.nn.relu(y2 * stats[6:7, :] + stats[7:8, :])

        @pl.when(k == 0)
        def _first():
            out[...] = h2

        @pl.when(k > 0)
        def _rest():
            out[...] = jnp.maximum(out[...], h2)


def _edge_conv(hs, g2, w1, b1, g1, be1, w2, b2, g2w, be2):
    w1a = w1[:EC, :]
    w1b = w1[EC:, :]
    row = lambda v: v.reshape(1, -1)
    return pl.pallas_call(
        _edge_body,
        grid=(3, K),
        in_specs=[
            pl.BlockSpec((N, EC), lambda p, k: (0, 0)),
            pl.BlockSpec((1, N, 16), lambda p, k: (k, 0, 0)),
            pl.BlockSpec((EC, HID), lambda p, k: (0, 0)),
            pl.BlockSpec((EC, HID), lambda p, k: (0, 0)),
            pl.BlockSpec((1, HID), lambda p, k: (0, 0)),
            pl.BlockSpec((1, HID), lambda p, k: (0, 0)),
            pl.BlockSpec((1, HID), lambda p, k: (0, 0)),
            pl.BlockSpec((HID, HID), lambda p, k: (0, 0)),
            pl.BlockSpec((1, HID), lambda p, k: (0, 0)),
            pl.BlockSpec((1, HID), lambda p, k: (0, 0)),
            pl.BlockSpec((1, HID), lambda p, k: (0, 0)),
        ],
        out_specs=pl.BlockSpec((N, HID), lambda p, k: (0, 0)),
        out_shape=jax.ShapeDtypeStruct((N, HID), F32),
        scratch_shapes=[pltpu.VMEM((8, HID), F32)],
    )(hs, g2, w1a, w1b, row(b1), row(g1), row(be1), w2, row(b2), row(g2w),
      row(be2))


def _cls_body(os_, op_, wca, wcb, bc1, gc, bec, wc2, bc2, out):
    y = (lax.dot_general(os_[...], wca[...], (((1,), (0,)), ((), ())),
                         preferred_element_type=F32)
         + lax.dot_general(op_[...], wcb[...], (((1,), (0,)), ((), ())),
                           preferred_element_type=F32)
         + bc1[...])
    m = jnp.mean(y, axis=0, keepdims=True)
    v = jnp.mean((y - m) ** 2, axis=0, keepdims=True)
    h = jax.nn.relu((y - m) / jnp.sqrt(v + EPS) * gc[...] + bec[...])
    out[...] = lax.dot_general(h, wc2[...], (((1,), (0,)), ((), ())),
                               preferred_element_type=F32) + bc2[...]


def _classifier(o_s, o_p, wc1, bc1, gc, bec, wc2, bc2):
    row = lambda v: v.reshape(1, -1)
    return pl.pallas_call(
        _cls_body,
        out_shape=jax.ShapeDtypeStruct((N, 1), F32),
    )(o_s, o_p, wc1[:HID, :], wc1[HID:, :], row(bc1), row(gc), row(bec),
      wc2, row(bc2))


def _bn_host(x, g, b):
    m = jnp.mean(x, axis=0, keepdims=True)
    v = jnp.mean((x - m) ** 2, axis=0, keepdims=True)
    return (x - m) / jnp.sqrt(v + EPS) * g + b


def _gather_edges(feat, idx):
    try:
        info = plsc.get_sparse_core_info()
        nc, ns = info.num_cores, info.num_subcores
    except Exception:
        nc, ns = 2, 16
    nw = nc * ns
    align = 8 * nw
    b_pad = ((_NEDGE + align - 1) // align) * align
    b_per_w = b_pad // nw
    table = jnp.concatenate([feat, jnp.zeros((N, 16 - EC), F32)], axis=1)
    idx_flat = idx.T.reshape(-1)
    idx_pad = jnp.concatenate(
        [idx_flat, jnp.zeros((b_pad - _NEDGE,), jnp.int32)])
    g = _sc_gather(table, idx_pad, b_pad, b_per_w, nc)
    return lax.slice(g, (0, 0), (_NEDGE, 16)).reshape(K, N, 16)


def kernel(x, batch, g_sn, b_sn, g_tn, b_tn, W_se, bi_se, g_se, be_se, W_pe,
           bi_pe, g_pe, be_pe, W1s, b1s, g1s, be1s, W2s, b2s, g2s, be2s, W1p,
           b1p, g1p, be1p, W2p, b2p, g2p, be2p, Wc1, bc1, gc, bec, Wc2, bc2):
    pos = x[:, :3]
    xs = _bn_host(pos, g_sn, b_sn)
    xt = _bn_host(x[:, 3:4], g_tn, b_tn)
    xp = jnp.concatenate([xs, xt], axis=1)
    hs = jax.nn.relu(_bn_host(xs @ W_se + bi_se, g_se, be_se))
    hp = jax.nn.relu(_bn_host(xp @ W_pe + bi_pe, g_pe, be_pe))

    batch_row = batch.reshape(N, 1)
    batch_col = batch.reshape(1, N)
    lo = batch[:: _RB]
    hi = batch[_RB - 1:: _RB]
    starts = jnp.searchsorted(batch, lo, side="left").astype(jnp.int32)
    ends = jnp.searchsorted(batch, hi, side="right").astype(jnp.int32)
    bounds = jnp.stack([starts, ends], axis=1)
    idx_s, idx_p = _knn2(hs, hp, batch_row, batch_col, bounds)

    g2_s = _gather_edges(hs, idx_s)
    g2_p = _gather_edges(hp, idx_p)

    o_s = _edge_conv(hs, g2_s, W1s, b1s, g1s, be1s, W2s, b2s, g2s, be2s)
    o_p = _edge_conv(hp, g2_p, W1p, b1p, g1p, be1p, W2p, b2p, g2p, be2p)

    return _classifier(o_s, o_p, Wc1, bc1, gc, bec, Wc2, bc2)
